# baseline (speedup 1.0000x reference)
import functools

import jax
import jax.numpy as jnp
from jax import lax
from jax.experimental import pallas as pl
from jax.experimental.pallas import tpu as pltpu
from jax.experimental.pallas import tpu_sc as plsc


def kernel(tokenized_prompts, ctx, token_embedding):
    n_cls, seq = tokenized_prompts.shape
    n_ctx, d = ctx.shape
    vocab = token_embedding.shape[0]
    n_rows = n_cls * seq

    table = jnp.concatenate([token_embedding, ctx], axis=0)
    pos = jnp.arange(seq, dtype=jnp.int32)[None, :]
    ctx_ids = (vocab - 1 + pos).astype(jnp.int32)
    idx = jnp.where((pos >= 1) & (pos < 1 + n_ctx),
                    ctx_ids, tokenized_prompts).reshape(-1)

    info = plsc.get_sparse_core_info()
    nc, ns = info.num_cores, info.num_subcores
    nw = nc * ns
    per_w = n_rows // nw
    chunk = 64
    n_chunks = per_w // chunk
    nbuf = 3

    mesh = plsc.VectorSubcoreMesh(core_axis_name="c", subcore_axis_name="s")

    @functools.partial(
        pl.kernel,
        out_type=jax.ShapeDtypeStruct((n_rows, d), jnp.float32),
        mesh=mesh,
        scratch_types=[
            pltpu.VMEM((per_w,), jnp.int32),
            pltpu.VMEM((nbuf, chunk, d), jnp.float32),
            [pltpu.SemaphoreType.DMA] * nbuf,
            [pltpu.SemaphoreType.DMA] * nbuf,
        ],
    )
    def _gather_kernel(idx_hbm, table_hbm, out_hbm, idx_v, rows_v,
                       gsems, ssems):
        wid = lax.axis_index("s") * nc + lax.axis_index("c")
        base = wid * per_w

        pltpu.sync_copy(idx_hbm.at[pl.ds(base, per_w)], idx_v)

        def gather_desc(k, b):
            return pltpu.make_async_copy(
                table_hbm.at[idx_v.at[pl.ds(k * chunk, chunk)]],
                rows_v.at[b], gsems[b])

        def store_desc(k, b):
            return pltpu.make_async_copy(
                rows_v.at[b], out_hbm.at[pl.ds(base + k * chunk, chunk)],
                ssems[b])

        gather_desc(0, 0).start()

        @pl.loop(0, n_chunks, step=nbuf)
        def _body(n):
            for b in range(nbuf):
                k = n + b
                bn = (b + 1) % nbuf

                @pl.when(k < n_chunks)
                def _():
                    gather_desc(k, b).wait()
                    store_desc(k, b).start()

                @pl.when(k >= 2)
                def _():
                    store_desc(k - 2, bn).wait()

                @pl.when(k + 1 < n_chunks)
                def _():
                    gather_desc(k + 1, bn).start()

        last_n = nbuf * ((n_chunks - 1) // nbuf)
        drained = last_n + nbuf - 3
        for k in range(max(0, drained + 1), n_chunks):
            store_desc(k, k % nbuf).wait()

    out = _gather_kernel(idx, table)
    return out.reshape(n_cls, seq, d)


# speedup vs baseline: 1.5076x; 1.5076x over previous
import functools

import jax
import jax.numpy as jnp
from jax import lax
from jax.experimental import pallas as pl
from jax.experimental.pallas import tpu as pltpu
from jax.experimental.pallas import tpu_sc as plsc


def kernel(tokenized_prompts, ctx, token_embedding):
    n_cls, seq = tokenized_prompts.shape
    n_ctx, d = ctx.shape
    vocab = token_embedding.shape[0]

    table = jnp.concatenate([token_embedding, ctx], axis=0)
    pos = jnp.arange(seq, dtype=jnp.int32)[None, :]
    ctx_ids = (vocab - 1 + pos).astype(jnp.int32)
    idx = jnp.where((pos >= 1) & (pos < 1 + n_ctx), ctx_ids,
                    tokenized_prompts)

    info = plsc.get_sparse_core_info()
    nc, ns = info.num_cores, info.num_subcores
    nw = nc * ns
    per_w = n_cls // nw
    nbuf = 2

    mesh = plsc.VectorSubcoreMesh(core_axis_name="c", subcore_axis_name="s")

    @functools.partial(
        pl.kernel,
        out_type=jax.ShapeDtypeStruct((n_cls, seq, d), jnp.float32),
        mesh=mesh,
        scratch_types=[
            pltpu.VMEM((per_w, seq), jnp.int32),
            pltpu.VMEM((nbuf, seq, d), jnp.float32),
            [pltpu.SemaphoreType.DMA] * nbuf,
            [pltpu.SemaphoreType.DMA] * nbuf,
        ],
    )
    def _gather_kernel(idx_hbm, table_hbm, out_hbm, idx_v, rows_v,
                       gsems, ssems):
        wid = lax.axis_index("s") * nc + lax.axis_index("c")
        base = wid * per_w

        pltpu.sync_copy(idx_hbm.at[pl.ds(base, per_w)], idx_v)

        def gather_desc(k, b):
            return pltpu.make_async_copy(
                table_hbm.at[idx_v.at[k]], rows_v.at[b], gsems[b])

        def store_desc(k, b):
            return pltpu.make_async_copy(
                rows_v.at[b], out_hbm.at[base + k], ssems[b])

        gather_desc(0, 0).start()

        @pl.loop(0, per_w, step=nbuf)
        def _body(n):
            for b in range(nbuf):
                k = n + b
                bn = (b + 1) % nbuf

                @pl.when(k < per_w)
                def _():
                    gather_desc(k, b).wait()
                    store_desc(k, b).start()

                @pl.when(k >= 2)
                def _():
                    store_desc(k - 2, bn).wait()

                @pl.when(k + 1 < per_w)
                def _():
                    gather_desc(k + 1, bn).start()

        last_n = nbuf * ((per_w - 1) // nbuf)
        drained = last_n + nbuf - 3
        for k in range(max(0, drained + 1), per_w):
            store_desc(k, k % nbuf).wait()

    return _gather_kernel(idx, table)
